# baseline (speedup 1.0000x reference)
import functools
import math

import numpy as np

import jax
import jax.numpy as jnp
from jax import lax
from jax.experimental import pallas as pl
from jax.experimental.pallas import tpu as pltpu
import jax.experimental.pallas.tpu_sc as plsc

_N = 10000
_E = 320000
_D = 128
_G = 64
_NC, _NS = 2, 16
_NW = _NC * _NS
_K = 80
_EPW = _E // _NW
_NCH = _EPW // _K
_RPT = _N // _NS
_ZB = 1000
_RING = 4
_NA = 10240
_BNS = 1.0 / math.sqrt(1.0 + 1e-5)
_SQ = float(np.sqrt(np.float32(1.0 + 1e-5)))

_BM = 2000
_NB = _N // _BM

_sc_mesh = plsc.VectorSubcoreMesh(
    core_axis_name="c", subcore_axis_name="s", num_cores=_NC, num_subcores=_NS
)


def _fill1d(ref, n, val):
    def _st(i, carry):
        ref[pl.ds(i * 16, 16)] = jnp.full((16,), val, jnp.float32)
        return carry

    lax.fori_loop(0, n // 16, _st, 0)
    if n % 16:
        ref[pl.ds(n - 16, 16)] = jnp.full((16,), val, jnp.float32)


def _deg_body(dst_hbm, degp_hbm, dacc, didx, ones_v, dbuf):
    c = lax.axis_index("c")
    s = lax.axis_index("s")
    wid = c * _NS + s

    _fill1d(ones_v, _K, 1.0)
    _fill1d(dbuf, _ZB, 0.0)

    @pl.when(s < _N // _ZB)
    def _():
        pltpu.sync_copy(dbuf, dacc.at[pl.ds(s * _ZB, _ZB)])

    plsc.subcore_barrier()

    base = wid * _EPW

    def _step(i, carry):
        pltpu.sync_copy(dst_hbm.at[pl.ds(base + i * _K, _K)], didx)
        pltpu.sync_copy(ones_v, dacc.at[didx], add=True)
        return carry

    lax.fori_loop(0, _NCH, _step, 0)
    plsc.subcore_barrier()

    @pl.when(s < _N // _ZB)
    def _():
        pltpu.sync_copy(dacc.at[pl.ds(s * _ZB, _ZB)], dbuf)
        pltpu.sync_copy(dbuf, degp_hbm.at[pl.ds(c * _N + s * _ZB, _ZB)])


_deg_kernel = functools.partial(
    pl.kernel,
    out_type=jax.ShapeDtypeStruct((_NC * _N,), jnp.float32),
    mesh=_sc_mesh,
    scratch_types=[
        pltpu.VMEM_SHARED((_N,), jnp.float32),
        pltpu.VMEM((_K,), jnp.int32),
        pltpu.VMEM((_K,), jnp.float32),
        pltpu.VMEM((_ZB,), jnp.float32),
    ],
)(_deg_body)


def _scale_rows(rows, dvs, dvd):
    def _nm(i, carry):
        dvs[pl.ds(i * 16, 16)] = dvs[pl.ds(i * 16, 16)] * dvd[pl.ds(i * 16, 16)]
        return carry

    lax.fori_loop(0, _K // 16, _nm, 0)

    def _sc(g, carry):
        nv = dvs[pl.ds(g * 16, 16)]
        for l in range(16):
            e = g * 16 + l
            bc = jnp.full((16,), nv[l], jnp.float32)
            for j in range(_D // 16):
                rows[e, pl.ds(j * 16, 16)] = rows[e, pl.ds(j * 16, 16)] * bc
        return carry

    lax.fori_loop(0, _K // 16, _sc, 0)


def _spmm_body(u_hbm, src_hbm, dst_hbm, dinv_hbm, sp_hbm, sacc,
               si0, si1, si2, si3, di0, di1, di2, di3,
               r0, r1, r2, r3, n0, n1, n2, n3, m0, m1, m2, m3,
               a0, a1, a2, a3, b0, b1, b2, b3, g0, g1, g2, g3,
               w0, w1, w2, w3, x0, x1, x2, x3, y0, y1, y2, y3):
    c = lax.axis_index("c")
    s = lax.axis_index("s")
    wid = c * _NS + s
    sidx = [si0, si1, si2, si3]
    didx = [di0, di1, di2, di3]
    rows = [r0, r1, r2, r3]
    nvs = [n0, n1, n2, n3]
    nvd = [m0, m1, m2, m3]
    isem = [a0, a1, a2, a3]
    jsem = [b0, b1, b2, b3]
    gsem = [g0, g1, g2, g3]
    wsem = [w0, w1, w2, w3]
    xsem = [x0, x1, x2, x3]
    ysem = [y0, y1, y2, y3]

    def _zf(i, carry):
        for j in range(_D // 16):
            r0[i, pl.ds(j * 16, 16)] = jnp.zeros((16,), jnp.float32)
        return carry

    lax.fori_loop(0, _K, _zf, 0)
    sd = []
    for k in range(_NA // _NS // _K):
        rr = (s * (_NA // _NS // _K) + k) * _K
        sd.append(pltpu.async_copy(r0, sacc.at[pl.ds(rr, _K)], w0))
    for d in sd:
        d.wait()
    plsc.subcore_barrier()

    ebase = wid * _EPW

    @pl.loop(0, _NCH - 1, step=_RING)
    def _ring(j0):
        idesc = []
        for b in range(_RING):
            off = ebase + (j0 + b) * _K
            idesc.append((
                pltpu.async_copy(src_hbm.at[pl.ds(off, _K)], sidx[b], isem[b]),
                pltpu.async_copy(dst_hbm.at[pl.ds(off, _K)], didx[b], jsem[b]),
            ))
        gdesc = []
        for b in range(_RING):
            idesc[b][0].wait()
            idesc[b][1].wait()
            gdesc.append((
                pltpu.async_copy(u_hbm.at[sidx[b]], rows[b], gsem[b]),
                pltpu.async_copy(dinv_hbm.at[sidx[b]], nvs[b], xsem[b]),
                pltpu.async_copy(dinv_hbm.at[didx[b]], nvd[b], ysem[b]),
            ))
        for b in range(_RING):
            for d in gdesc[b]:
                d.wait()
            _scale_rows(rows[b], nvs[b], nvd[b])
            pltpu.sync_copy(rows[b], sacc.at[didx[b]], add=True)

    toff = ebase + (_NCH - 1) * _K
    pltpu.sync_copy(src_hbm.at[pl.ds(toff, _K)], sidx[0])
    pltpu.sync_copy(dst_hbm.at[pl.ds(toff, _K)], didx[0])
    pltpu.async_copy(u_hbm.at[sidx[0]], rows[0], gsem[0]).wait()
    pltpu.async_copy(dinv_hbm.at[sidx[0]], nvs[0], xsem[0]).wait()
    pltpu.async_copy(dinv_hbm.at[didx[0]], nvd[0], ysem[0]).wait()
    _scale_rows(rows[0], nvs[0], nvd[0])
    pltpu.sync_copy(rows[0], sacc.at[didx[0]], add=True)

    plsc.subcore_barrier()

    wd = [None] * _RING
    for k in range(_NA // _NS // _K):
        b = k % _RING
        rr = (s * (_NA // _NS // _K) + k) * _K
        if wd[b] is not None:
            wd[b].wait()
        pltpu.async_copy(sacc.at[pl.ds(rr, _K)], rows[b], gsem[b]).wait()
        wd[b] = pltpu.async_copy(rows[b], sp_hbm.at[c, pl.ds(rr, _K)], wsem[b])
    for d in wd:
        d.wait()


_spmm_kernel = functools.partial(
    pl.kernel,
    out_type=jax.ShapeDtypeStruct((_NC, _NA, _D), jnp.float32),
    mesh=_sc_mesh,
    scratch_types=(
        [pltpu.VMEM_SHARED((_NA, _D), jnp.float32)]
        + [pltpu.VMEM((_K,), jnp.int32)] * (2 * _RING)
        + [pltpu.VMEM((_K, _D), jnp.float32)] * _RING
        + [pltpu.VMEM((_K,), jnp.float32)] * (2 * _RING)
        + [pltpu.SemaphoreType.DMA] * (6 * _RING)
    ),
)(_spmm_body)


def _mm1_body(x_ref, w_ref, d0_ref, d1_ref, u_ref, dinv_ref):
    deg = d0_ref[0] + d1_ref[0] + 1.0
    dinv = lax.rsqrt(deg)
    dinv_ref[0] = dinv
    xx = x_ref[...]
    ww = w_ref[...]
    xh = xx.astype(jnp.bfloat16)
    xl = (xx - xh.astype(jnp.float32)).astype(jnp.bfloat16)
    wh = ww.astype(jnp.bfloat16)
    wl = (ww - wh.astype(jnp.float32)).astype(jnp.bfloat16)

    def _p(a, bb2):
        return jnp.dot(a, bb2, preferred_element_type=jnp.float32)

    u_ref[...] = _p(xh, wl) + _p(xl, wh) + _p(xh, wh)


def _mm1(x, W1, d0, d1):
    return pl.pallas_call(
        _mm1_body,
        grid=(_NB,),
        in_specs=[
            pl.BlockSpec((_BM, _D), lambda i: (i, 0)),
            pl.BlockSpec((_D, _D), lambda i: (0, 0)),
            pl.BlockSpec((1, _BM, 1), lambda i: (i, 0, 0)),
            pl.BlockSpec((1, _BM, 1), lambda i: (i, 0, 0)),
        ],
        out_specs=[
            pl.BlockSpec((_BM, _D), lambda i: (i, 0)),
            pl.BlockSpec((1, _BM, 1), lambda i: (i, 0, 0)),
        ],
        out_shape=[
            jax.ShapeDtypeStruct((_N, _D), jnp.float32),
            jax.ShapeDtypeStruct((_NB, _BM, 1), jnp.float32),
        ],
    )(x, W1, d0, d1)


def _mid_body(s0_ref, s1_ref, u_ref, dinv_ref, b_ref, g_ref, be_ref, w_ref,
              out_ref):
    dinv = dinv_ref[0]
    t = (s0_ref[...] + s1_ref[...] + u_ref[...] * (dinv * dinv)) + b_ref[...]
    h = jnp.maximum(t / _SQ * g_ref[...] + be_ref[...], 0.0)
    out_ref[...] = jnp.dot(h.astype(jnp.bfloat16),
                           w_ref[...].astype(jnp.bfloat16),
                           preferred_element_type=jnp.float32)


def _mid(s0, s1, u, dinv, b, g, be, W):
    return pl.pallas_call(
        _mid_body,
        grid=(_NB,),
        in_specs=[
            pl.BlockSpec((_BM, _D), lambda i: (i, 0)),
            pl.BlockSpec((_BM, _D), lambda i: (i, 0)),
            pl.BlockSpec((_BM, _D), lambda i: (i, 0)),
            pl.BlockSpec((1, _BM, 1), lambda i: (i, 0, 0)),
            pl.BlockSpec((1, _D), lambda i: (0, 0)),
            pl.BlockSpec((1, _D), lambda i: (0, 0)),
            pl.BlockSpec((1, _D), lambda i: (0, 0)),
            pl.BlockSpec((_D, _D), lambda i: (0, 0)),
        ],
        out_specs=pl.BlockSpec((_BM, _D), lambda i: (i, 0)),
        out_shape=jax.ShapeDtypeStruct((_N, _D), jnp.float32),
    )(s0, s1, u, dinv, b, g, be, W)


def _head_body(s0_ref, s1_ref, u_ref, dinv_ref, b_ref, g_ref, be_ref,
               batch_ref, m1w, m1b, m2w, m2b, m3w, m3b, m4w, m4b,
               out_ref, pacc, cacc):
    i = pl.program_id(0)

    @pl.when(i == 0)
    def _():
        pacc[...] = jnp.zeros_like(pacc)
        cacc[...] = jnp.zeros_like(cacc)

    gids = lax.broadcasted_iota(jnp.int32, (_G, 1), 0)

    def _slab(k, carry):
        acc, cc = carry
        r = k * 8
        dinv = dinv_ref[0, pl.ds(r, 8), :]
        t = (s0_ref[pl.ds(r, 8), :] + s1_ref[pl.ds(r, 8), :]
             + u_ref[pl.ds(r, 8), :] * (dinv * dinv)) + b_ref[...]
        h = jnp.maximum(t / _SQ * g_ref[...] + be_ref[...], 0.0)
        bb = batch_ref[0, pl.ds(k, 1), :]
        oh = jnp.where(bb == gids, 1.0, 0.0)
        acc = acc + jnp.dot(oh, h, preferred_element_type=jnp.float32,
                            precision=lax.Precision.HIGHEST)
        cc = cc + jnp.sum(oh, axis=1, keepdims=True)
        return acc, cc

    acc, cc = lax.fori_loop(0, _BM // 8, _slab, (pacc[...], cacc[...]))
    pacc[...] = acc
    cacc[...] = cc

    @pl.when(i == _NB - 1)
    def _():
        pooled = pacc[...] / jnp.maximum(cacc[...], 1.0)
        def _bdot(a, w):
            return jnp.dot(a.astype(jnp.bfloat16), w.astype(jnp.bfloat16),
                           preferred_element_type=jnp.float32)
        z = jnp.maximum(_bdot(pooled, m1w[...]) + m1b[...], 0.0)
        z = jnp.maximum(_bdot(z, m2w[...]) + m2b[...], 0.0)
        z = jnp.maximum(_bdot(z, m3w[...]) + m3b[...], 0.0)
        out_ref[...] = _bdot(z, m4w[...]) + m4b[...]


def _head(s0, s1, u, dinv, b, g, be, batchR, M1w, M1b, M2w, M2b, M3w, M3b,
          M4w, M4b):
    wspec = lambda: pl.BlockSpec(None, lambda i: (0, 0))
    return pl.pallas_call(
        _head_body,
        grid=(_NB,),
        in_specs=[
            pl.BlockSpec((_BM, _D), lambda i: (i, 0)),
            pl.BlockSpec((_BM, _D), lambda i: (i, 0)),
            pl.BlockSpec((_BM, _D), lambda i: (i, 0)),
            pl.BlockSpec((1, _BM, 1), lambda i: (i, 0, 0)),
            pl.BlockSpec((1, _D), lambda i: (0, 0)),
            pl.BlockSpec((1, _D), lambda i: (0, 0)),
            pl.BlockSpec((1, _D), lambda i: (0, 0)),
            pl.BlockSpec((1, _BM // 8, 8), lambda i: (i, 0, 0)),
            pl.BlockSpec((_D, _D), lambda i: (0, 0)),
            pl.BlockSpec((1, _D), lambda i: (0, 0)),
            pl.BlockSpec((_D, _G), lambda i: (0, 0)),
            pl.BlockSpec((1, _G), lambda i: (0, 0)),
            pl.BlockSpec((_G, 32), lambda i: (0, 0)),
            pl.BlockSpec((1, 32), lambda i: (0, 0)),
            pl.BlockSpec((32, 2), lambda i: (0, 0)),
            pl.BlockSpec((1, 2), lambda i: (0, 0)),
        ],
        out_specs=pl.BlockSpec((_G, 2), lambda i: (0, 0)),
        out_shape=jax.ShapeDtypeStruct((_G, 2), jnp.float32),
        scratch_shapes=[
            pltpu.VMEM((_G, _D), jnp.float32),
            pltpu.VMEM((_G, 1), jnp.float32),
        ],
    )(s0, s1, u, dinv, b, g, be, batchR, M1w, M1b, M2w, M2b, M3w, M3b,
      M4w, M4b)


def kernel(x, edge_index, batch, W1, b1, g1, be1, W2, b2, g2, be2,
           W3, b3, g3, be3, M1w, M1b, M2w, M2b, M3w, M3b, M4w, M4b):
    src = edge_index[0].astype(jnp.int32)
    dst = edge_index[1].astype(jnp.int32)
    perm = jnp.argsort(dst, stable=True)
    srcs = src[perm]
    dsts = dst[perm]


    degp = _deg_kernel(dst)
    d0 = degp[:_N].reshape(_NB, _BM, 1)
    d1 = degp[_N:].reshape(_NB, _BM, 1)

    u1, dinvR = _mm1(x, W1, d0, d1)
    dinvF = dinvR.reshape(_N)

    sp1 = _spmm_kernel(u1, srcs, dsts, dinvF)[:, :_N]
    u2 = _mid(sp1[0], sp1[1], u1, dinvR, b1.reshape(1, _D),
              g1.reshape(1, _D), be1.reshape(1, _D), W2)

    sp2 = _spmm_kernel(u2, srcs, dsts, dinvF)[:, :_N]
    u3 = _mid(sp2[0], sp2[1], u2, dinvR, b2.reshape(1, _D),
              g2.reshape(1, _D), be2.reshape(1, _D), W3)

    sp3 = _spmm_kernel(u3, srcs, dsts, dinvF)[:, :_N]
    out = _head(sp3[0], sp3[1], u3, dinvR, b3.reshape(1, _D),
                g3.reshape(1, _D), be3.reshape(1, _D),
                batch.astype(jnp.int32).reshape(_NB, _BM // 8, 8),
                M1w, M1b.reshape(1, _D), M2w, M2b.reshape(1, _G),
                M3w, M3b.reshape(1, 32), M4w, M4b.reshape(1, 2))
    return out


# speedup vs baseline: 1.3304x; 1.3304x over previous
import functools
import math

import numpy as np

import jax
import jax.numpy as jnp
from jax import lax
from jax.experimental import pallas as pl
from jax.experimental.pallas import tpu as pltpu
import jax.experimental.pallas.tpu_sc as plsc

_N = 10000
_E = 320000
_D = 128
_G = 64
_NC, _NS = 2, 16
_NW = _NC * _NS
_K = 80
_EPW = _E // _NW
_NCH = _EPW // _K
_RPT = _N // _NS
_ZB = 1000
_RING = 4
_NA = 10240
_BNS = 1.0 / math.sqrt(1.0 + 1e-5)
_SQ = float(np.sqrt(np.float32(1.0 + 1e-5)))

_BM = 2000
_NB = _N // _BM

_sc_mesh = plsc.VectorSubcoreMesh(
    core_axis_name="c", subcore_axis_name="s", num_cores=_NC, num_subcores=_NS
)


def _fill1d(ref, n, val):
    def _st(i, carry):
        ref[pl.ds(i * 16, 16)] = jnp.full((16,), val, jnp.float32)
        return carry

    lax.fori_loop(0, n // 16, _st, 0)
    if n % 16:
        ref[pl.ds(n - 16, 16)] = jnp.full((16,), val, jnp.float32)


def _deg_body(dst_hbm, degp_hbm, dacc, didx, ones_v, dbuf):
    c = lax.axis_index("c")
    s = lax.axis_index("s")
    wid = c * _NS + s

    _fill1d(ones_v, _K, 1.0)
    _fill1d(dbuf, _ZB, 0.0)

    @pl.when(s < _N // _ZB)
    def _():
        pltpu.sync_copy(dbuf, dacc.at[pl.ds(s * _ZB, _ZB)])

    plsc.subcore_barrier()

    base = wid * _EPW

    def _step(i, carry):
        pltpu.sync_copy(dst_hbm.at[pl.ds(base + i * _K, _K)], didx)
        pltpu.sync_copy(ones_v, dacc.at[didx], add=True)
        return carry

    lax.fori_loop(0, _NCH, _step, 0)
    plsc.subcore_barrier()

    @pl.when(s < _N // _ZB)
    def _():
        pltpu.sync_copy(dacc.at[pl.ds(s * _ZB, _ZB)], dbuf)
        pltpu.sync_copy(dbuf, degp_hbm.at[pl.ds(c * _N + s * _ZB, _ZB)])


_deg_kernel = functools.partial(
    pl.kernel,
    out_type=jax.ShapeDtypeStruct((_NC * _N,), jnp.float32),
    mesh=_sc_mesh,
    scratch_types=[
        pltpu.VMEM_SHARED((_N,), jnp.float32),
        pltpu.VMEM((_K,), jnp.int32),
        pltpu.VMEM((_K,), jnp.float32),
        pltpu.VMEM((_ZB,), jnp.float32),
    ],
)(_deg_body)


def _scale_rows(rows, dvs, dvd):
    def _nm(i, carry):
        dvs[pl.ds(i * 16, 16)] = dvs[pl.ds(i * 16, 16)] * dvd[pl.ds(i * 16, 16)]
        return carry

    lax.fori_loop(0, _K // 16, _nm, 0)

    def _sc(g, carry):
        nv = dvs[pl.ds(g * 16, 16)]
        for l in range(16):
            e = g * 16 + l
            bc = jnp.full((16,), nv[l], jnp.float32)
            for j in range(_D // 16):
                rows[e, pl.ds(j * 16, 16)] = rows[e, pl.ds(j * 16, 16)] * bc
        return carry

    lax.fori_loop(0, _K // 16, _sc, 0)


def _spmm_body(u_hbm, src_hbm, dst_hbm, dinv_hbm, sp_hbm, sacc,
               si0, si1, si2, si3, di0, di1, di2, di3,
               r0, r1, r2, r3, n0, n1, n2, n3, m0, m1, m2, m3,
               a0, a1, a2, a3, b0, b1, b2, b3, g0, g1, g2, g3,
               w0, w1, w2, w3, x0, x1, x2, x3, y0, y1, y2, y3):
    c = lax.axis_index("c")
    s = lax.axis_index("s")
    wid = c * _NS + s
    sidx = [si0, si1, si2, si3]
    didx = [di0, di1, di2, di3]
    rows = [r0, r1, r2, r3]
    nvs = [n0, n1, n2, n3]
    nvd = [m0, m1, m2, m3]
    isem = [a0, a1, a2, a3]
    jsem = [b0, b1, b2, b3]
    gsem = [g0, g1, g2, g3]
    wsem = [w0, w1, w2, w3]
    xsem = [x0, x1, x2, x3]
    ysem = [y0, y1, y2, y3]

    def _zf(i, carry):
        for j in range(_D // 16):
            r0[i, pl.ds(j * 16, 16)] = jnp.zeros((16,), jnp.float32)
        return carry

    lax.fori_loop(0, _K, _zf, 0)
    sd = []
    for k in range(_NA // _NS // _K):
        rr = (s * (_NA // _NS // _K) + k) * _K
        sd.append(pltpu.async_copy(r0, sacc.at[pl.ds(rr, _K)], w0))
    for d in sd:
        d.wait()
    plsc.subcore_barrier()

    ebase = wid * _EPW

    @pl.loop(0, _NCH - 1, step=_RING)
    def _ring(j0):
        idesc = []
        for b in range(_RING):
            off = ebase + (j0 + b) * _K
            idesc.append((
                pltpu.async_copy(src_hbm.at[pl.ds(off, _K)], sidx[b], isem[b]),
                pltpu.async_copy(dst_hbm.at[pl.ds(off, _K)], didx[b], jsem[b]),
            ))
        gdesc = []
        for b in range(_RING):
            idesc[b][0].wait()
            idesc[b][1].wait()
            gdesc.append((
                pltpu.async_copy(u_hbm.at[sidx[b]], rows[b], gsem[b]),
                pltpu.async_copy(dinv_hbm.at[sidx[b]], nvs[b], xsem[b]),
                pltpu.async_copy(dinv_hbm.at[didx[b]], nvd[b], ysem[b]),
            ))
        for b in range(_RING):
            for d in gdesc[b]:
                d.wait()
            _scale_rows(rows[b], nvs[b], nvd[b])
            pltpu.sync_copy(rows[b], sacc.at[didx[b]], add=True)

    toff = ebase + (_NCH - 1) * _K
    pltpu.sync_copy(src_hbm.at[pl.ds(toff, _K)], sidx[0])
    pltpu.sync_copy(dst_hbm.at[pl.ds(toff, _K)], didx[0])
    pltpu.async_copy(u_hbm.at[sidx[0]], rows[0], gsem[0]).wait()
    pltpu.async_copy(dinv_hbm.at[sidx[0]], nvs[0], xsem[0]).wait()
    pltpu.async_copy(dinv_hbm.at[didx[0]], nvd[0], ysem[0]).wait()
    _scale_rows(rows[0], nvs[0], nvd[0])
    pltpu.sync_copy(rows[0], sacc.at[didx[0]], add=True)

    plsc.subcore_barrier()

    wd = [None] * _RING
    for k in range(_NA // _NS // _K):
        b = k % _RING
        rr = (s * (_NA // _NS // _K) + k) * _K
        if wd[b] is not None:
            wd[b].wait()
        pltpu.async_copy(sacc.at[pl.ds(rr, _K)], rows[b], gsem[b]).wait()
        wd[b] = pltpu.async_copy(rows[b], sp_hbm.at[c, pl.ds(rr, _K)], wsem[b])
    for d in wd:
        d.wait()


_spmm_kernel = functools.partial(
    pl.kernel,
    out_type=jax.ShapeDtypeStruct((_NC, _NA, _D), jnp.float32),
    mesh=_sc_mesh,
    scratch_types=(
        [pltpu.VMEM_SHARED((_NA, _D), jnp.float32)]
        + [pltpu.VMEM((_K,), jnp.int32)] * (2 * _RING)
        + [pltpu.VMEM((_K, _D), jnp.float32)] * _RING
        + [pltpu.VMEM((_K,), jnp.float32)] * (2 * _RING)
        + [pltpu.SemaphoreType.DMA] * (6 * _RING)
    ),
)(_spmm_body)


def _mm1_body(x_ref, w_ref, d0_ref, d1_ref, u_ref, dinv_ref):
    deg = d0_ref[0] + d1_ref[0] + 1.0
    dinv = lax.rsqrt(deg)
    dinv_ref[0] = dinv
    xx = x_ref[...]
    ww = w_ref[...]
    xh = xx.astype(jnp.bfloat16)
    xl = (xx - xh.astype(jnp.float32)).astype(jnp.bfloat16)
    wh = ww.astype(jnp.bfloat16)
    wl = (ww - wh.astype(jnp.float32)).astype(jnp.bfloat16)

    def _p(a, bb2):
        return jnp.dot(a, bb2, preferred_element_type=jnp.float32)

    u_ref[...] = _p(xh, wl) + _p(xl, wh) + _p(xh, wh)


def _mm1(x, W1, d0, d1):
    return pl.pallas_call(
        _mm1_body,
        grid=(_NB,),
        in_specs=[
            pl.BlockSpec((_BM, _D), lambda i: (i, 0)),
            pl.BlockSpec((_D, _D), lambda i: (0, 0)),
            pl.BlockSpec((1, _BM, 1), lambda i: (i, 0, 0)),
            pl.BlockSpec((1, _BM, 1), lambda i: (i, 0, 0)),
        ],
        out_specs=[
            pl.BlockSpec((_BM, _D), lambda i: (i, 0)),
            pl.BlockSpec((1, _BM, 1), lambda i: (i, 0, 0)),
        ],
        out_shape=[
            jax.ShapeDtypeStruct((_N, _D), jnp.float32),
            jax.ShapeDtypeStruct((_NB, _BM, 1), jnp.float32),
        ],
    )(x, W1, d0, d1)


def _mid_body(s0_ref, s1_ref, u_ref, dinv_ref, b_ref, g_ref, be_ref, w_ref,
              out_ref):
    dinv = dinv_ref[0]
    t = (s0_ref[...] + s1_ref[...] + u_ref[...] * (dinv * dinv)) + b_ref[...]
    h = jnp.maximum(t / _SQ * g_ref[...] + be_ref[...], 0.0)
    out_ref[...] = jnp.dot(h.astype(jnp.bfloat16),
                           w_ref[...].astype(jnp.bfloat16),
                           preferred_element_type=jnp.float32)


def _mid(s0, s1, u, dinv, b, g, be, W):
    return pl.pallas_call(
        _mid_body,
        grid=(_NB,),
        in_specs=[
            pl.BlockSpec((_BM, _D), lambda i: (i, 0)),
            pl.BlockSpec((_BM, _D), lambda i: (i, 0)),
            pl.BlockSpec((_BM, _D), lambda i: (i, 0)),
            pl.BlockSpec((1, _BM, 1), lambda i: (i, 0, 0)),
            pl.BlockSpec((1, _D), lambda i: (0, 0)),
            pl.BlockSpec((1, _D), lambda i: (0, 0)),
            pl.BlockSpec((1, _D), lambda i: (0, 0)),
            pl.BlockSpec((_D, _D), lambda i: (0, 0)),
        ],
        out_specs=pl.BlockSpec((_BM, _D), lambda i: (i, 0)),
        out_shape=jax.ShapeDtypeStruct((_N, _D), jnp.float32),
    )(s0, s1, u, dinv, b, g, be, W)


def _head_body(s0_ref, s1_ref, u_ref, dinv_ref, b_ref, g_ref, be_ref,
               batch_ref, m1w, m1b, m2w, m2b, m3w, m3b, m4w, m4b,
               out_ref, pacc, cacc):
    i = pl.program_id(0)

    @pl.when(i == 0)
    def _():
        pacc[...] = jnp.zeros_like(pacc)
        cacc[...] = jnp.zeros_like(cacc)

    gids = lax.broadcasted_iota(jnp.int32, (_G, 1), 0)

    def _slab(k, carry):
        acc, cc = carry
        r = k * 8
        dinv = dinv_ref[0, pl.ds(r, 8), :]
        t = (s0_ref[pl.ds(r, 8), :] + s1_ref[pl.ds(r, 8), :]
             + u_ref[pl.ds(r, 8), :] * (dinv * dinv)) + b_ref[...]
        h = jnp.maximum(t / _SQ * g_ref[...] + be_ref[...], 0.0)
        bb = batch_ref[0, pl.ds(k, 1), :]
        oh = jnp.where(bb == gids, 1.0, 0.0)
        acc = acc + jnp.dot(oh, h, preferred_element_type=jnp.float32,
                            precision=lax.Precision.HIGHEST)
        cc = cc + jnp.sum(oh, axis=1, keepdims=True)
        return acc, cc

    acc, cc = lax.fori_loop(0, _BM // 8, _slab, (pacc[...], cacc[...]))
    pacc[...] = acc
    cacc[...] = cc

    @pl.when(i == _NB - 1)
    def _():
        pooled = pacc[...] / jnp.maximum(cacc[...], 1.0)
        def _bdot(a, w):
            return jnp.dot(a.astype(jnp.bfloat16), w.astype(jnp.bfloat16),
                           preferred_element_type=jnp.float32)
        z = jnp.maximum(_bdot(pooled, m1w[...]) + m1b[...], 0.0)
        z = jnp.maximum(_bdot(z, m2w[...]) + m2b[...], 0.0)
        z = jnp.maximum(_bdot(z, m3w[...]) + m3b[...], 0.0)
        out_ref[...] = _bdot(z, m4w[...]) + m4b[...]


def _head(s0, s1, u, dinv, b, g, be, batchR, M1w, M1b, M2w, M2b, M3w, M3b,
          M4w, M4b):
    wspec = lambda: pl.BlockSpec(None, lambda i: (0, 0))
    return pl.pallas_call(
        _head_body,
        grid=(_NB,),
        in_specs=[
            pl.BlockSpec((_BM, _D), lambda i: (i, 0)),
            pl.BlockSpec((_BM, _D), lambda i: (i, 0)),
            pl.BlockSpec((_BM, _D), lambda i: (i, 0)),
            pl.BlockSpec((1, _BM, 1), lambda i: (i, 0, 0)),
            pl.BlockSpec((1, _D), lambda i: (0, 0)),
            pl.BlockSpec((1, _D), lambda i: (0, 0)),
            pl.BlockSpec((1, _D), lambda i: (0, 0)),
            pl.BlockSpec((1, _BM // 8, 8), lambda i: (i, 0, 0)),
            pl.BlockSpec((_D, _D), lambda i: (0, 0)),
            pl.BlockSpec((1, _D), lambda i: (0, 0)),
            pl.BlockSpec((_D, _G), lambda i: (0, 0)),
            pl.BlockSpec((1, _G), lambda i: (0, 0)),
            pl.BlockSpec((_G, 32), lambda i: (0, 0)),
            pl.BlockSpec((1, 32), lambda i: (0, 0)),
            pl.BlockSpec((32, 2), lambda i: (0, 0)),
            pl.BlockSpec((1, 2), lambda i: (0, 0)),
        ],
        out_specs=pl.BlockSpec((_G, 2), lambda i: (0, 0)),
        out_shape=jax.ShapeDtypeStruct((_G, 2), jnp.float32),
        scratch_shapes=[
            pltpu.VMEM((_G, _D), jnp.float32),
            pltpu.VMEM((_G, 1), jnp.float32),
        ],
    )(s0, s1, u, dinv, b, g, be, batchR, M1w, M1b, M2w, M2b, M3w, M3b,
      M4w, M4b)


def kernel(x, edge_index, batch, W1, b1, g1, be1, W2, b2, g2, be2,
           W3, b3, g3, be3, M1w, M1b, M2w, M2b, M3w, M3b, M4w, M4b):
    src = edge_index[0].astype(jnp.int32)
    dst = edge_index[1].astype(jnp.int32)


    degp = _deg_kernel(dst)
    d0 = degp[:_N].reshape(_NB, _BM, 1)
    d1 = degp[_N:].reshape(_NB, _BM, 1)

    u1, dinvR = _mm1(x, W1, d0, d1)
    dinvF = dinvR.reshape(_N)

    sp1 = _spmm_kernel(u1, src, dst, dinvF)[:, :_N]
    u2 = _mid(sp1[0], sp1[1], u1, dinvR, b1.reshape(1, _D),
              g1.reshape(1, _D), be1.reshape(1, _D), W2)

    sp2 = _spmm_kernel(u2, src, dst, dinvF)[:, :_N]
    u3 = _mid(sp2[0], sp2[1], u2, dinvR, b2.reshape(1, _D),
              g2.reshape(1, _D), be2.reshape(1, _D), W3)

    sp3 = _spmm_kernel(u3, src, dst, dinvF)[:, :_N]
    out = _head(sp3[0], sp3[1], u3, dinvR, b3.reshape(1, _D),
                g3.reshape(1, _D), be3.reshape(1, _D),
                batch.astype(jnp.int32).reshape(_NB, _BM // 8, 8),
                M1w, M1b.reshape(1, _D), M2w, M2b.reshape(1, _G),
                M3w, M3b.reshape(1, 32), M4w, M4b.reshape(1, 2))
    return out
